# baseline (speedup 1.0000x reference)
import functools

import jax
import jax.numpy as jnp
from jax import lax
from jax.experimental import pallas as pl
from jax.experimental.pallas import tpu as pltpu
from jax.experimental.pallas import tpu_sc as plsc

_INTERPRET = False

F32 = jnp.float32


def _mm(a, b):
    return jnp.dot(a, b, preferred_element_type=F32)


def _node_enc_body(x_ref, wa1_ref, wa2_ref, wb1_ref, wb2_ref, ha_ref, hb_ref):
    x = x_ref[...]
    blk = x.shape[0]
    pad = jnp.zeros((blk, 3), F32)
    ha = jax.nn.relu(_mm(jax.nn.relu(_mm(x, wa1_ref[...])), wa2_ref[...]))
    hb = jax.nn.relu(_mm(jax.nn.relu(_mm(x, wb1_ref[...])), wb2_ref[...]))
    ha_ref[...] = jnp.concatenate([ha, pad], axis=1)
    hb_ref[...] = jnp.concatenate([hb, pad], axis=1)


def _node_encoders(x, p_ec, p_hc, blk):
    n = x.shape[0]
    grid = n // blk
    full = lambda s: pl.BlockSpec(s, lambda i: (0, 0))
    return pl.pallas_call(
        _node_enc_body,
        grid=(grid,),
        in_specs=[
            pl.BlockSpec((blk, x.shape[1]), lambda i: (i, 0)),
            full(p_ec[0]["W"].shape), full(p_ec[1]["W"].shape),
            full(p_hc[0]["W"].shape), full(p_hc[1]["W"].shape),
        ],
        out_specs=[
            pl.BlockSpec((blk, 8), lambda i: (i, 0)),
            pl.BlockSpec((blk, 8), lambda i: (i, 0)),
        ],
        out_shape=[
            jax.ShapeDtypeStruct((n, 8), F32),
            jax.ShapeDtypeStruct((n, 8), F32),
        ],
        interpret=_INTERPRET,
    )(x, p_ec[0]["W"], p_ec[1]["W"], p_hc[0]["W"], p_hc[1]["W"])


def _edge_enc_body(ea_ref, wa1_ref, wa2_ref, wb1_ref, wb2_ref, ea_out, eb_out):
    ea = ea_ref[...]
    ea_out[...] = jax.nn.relu(_mm(jax.nn.relu(_mm(ea, wa1_ref[...])), wa2_ref[...]))
    eb_out[...] = jax.nn.relu(_mm(jax.nn.relu(_mm(ea, wb1_ref[...])), wb2_ref[...]))


def _edge_encoders(edge_attr, p_ec, p_hc, blk):
    e = edge_attr.shape[0]
    grid = e // blk
    full = lambda s: pl.BlockSpec(s, lambda i: (0, 0))
    return pl.pallas_call(
        _edge_enc_body,
        grid=(grid,),
        in_specs=[
            pl.BlockSpec((blk, edge_attr.shape[1]), lambda i: (i, 0)),
            full(p_ec[0]["W"].shape), full(p_ec[1]["W"].shape),
            full(p_hc[0]["W"].shape), full(p_hc[1]["W"].shape),
        ],
        out_specs=[
            pl.BlockSpec((blk, 4), lambda i: (i, 0)),
            pl.BlockSpec((blk, 4), lambda i: (i, 0)),
        ],
        out_shape=[
            jax.ShapeDtypeStruct((e, 4), F32),
            jax.ShapeDtypeStruct((e, 4), F32),
        ],
        interpret=_INTERPRET,
    )(edge_attr, p_ec[0]["W"], p_ec[1]["W"], p_hc[0]["W"], p_hc[1]["W"])


def _ec_edge_body(gs_ref, gd_ref, ee_ref,
                  ws_ref, wd_ref, we_ref, b1_ref, w2_ref, b2_ref,
                  v1_ref, c1_ref, v2_ref, c2_ref, v3_ref, c3_ref,
                  w_out):
    hs = gs_ref[...][:, 0:5]
    hd = gd_ref[...][:, 0:5]
    ee = ee_ref[...]
    hid = jax.nn.relu(_mm(hs, ws_ref[...]) + _mm(hd, wd_ref[...])
                      + _mm(ee, we_ref[...]) + b1_ref[...])
    e2 = _mm(hid, w2_ref[...]) + b2_ref[...]
    t = jax.nn.relu(_mm(e2, v1_ref[...]) + c1_ref[...])
    t = jax.nn.relu(_mm(t, v2_ref[...]) + c2_ref[...])
    w_out[...] = jax.nn.sigmoid(_mm(t, v3_ref[...]) + c3_ref[...])


def _ec_edge(gs, gd, ee, phi_e, ec_w, blk):
    e = gs.shape[0]
    grid = e // blk
    w1 = phi_e[0]["W"]
    args = (
        gs, gd, ee,
        w1[0:5], w1[5:10], w1[10:14], phi_e[0]["b"].reshape(1, -1),
        phi_e[1]["W"], phi_e[1]["b"].reshape(1, -1),
        ec_w[0]["W"], ec_w[0]["b"].reshape(1, -1),
        ec_w[1]["W"], ec_w[1]["b"].reshape(1, -1),
        ec_w[2]["W"], ec_w[2]["b"].reshape(1, -1),
    )
    blk_specs = [
        pl.BlockSpec((blk, 8), lambda i: (i, 0)),
        pl.BlockSpec((blk, 8), lambda i: (i, 0)),
        pl.BlockSpec((blk, 4), lambda i: (i, 0)),
    ] + [pl.BlockSpec(a.shape, lambda i: (0, 0)) for a in args[3:]]
    return pl.pallas_call(
        _ec_edge_body,
        grid=(grid,),
        in_specs=blk_specs,
        out_specs=pl.BlockSpec((blk, 1), lambda i: (i, 0)),
        out_shape=jax.ShapeDtypeStruct((e, 1), F32),
        interpret=_INTERPRET,
    )(*args)


def _phi_e_body(gs_ref, gd_ref, el_ref, w_ref,
                ws_ref, wd_ref, we_ref, b1_ref, w2_ref, b2_ref,
                enext_out, msg_out):
    hs = gs_ref[...][:, 0:5]
    hd = gd_ref[...][:, 0:5]
    el = el_ref[...]
    blk = hs.shape[0]
    hid = jax.nn.relu(_mm(hs, ws_ref[...]) + _mm(hd, wd_ref[...])
                      + _mm(el, we_ref[...]) + b1_ref[...])
    e_new = _mm(hid, w2_ref[...]) + b2_ref[...]
    mask = (w_ref[...] > 0.5).astype(F32)
    msg = e_new * mask
    enext_out[...] = el + e_new
    msg_out[...] = jnp.concatenate([msg, jnp.zeros((blk, 4), F32)], axis=1)


def _hc_phi_e(gs, gd, el, w, phi_e, blk):
    e = gs.shape[0]
    grid = e // blk
    w1 = phi_e[0]["W"]
    args = (
        gs, gd, el, w,
        w1[0:5], w1[5:10], w1[10:14], phi_e[0]["b"].reshape(1, -1),
        phi_e[1]["W"], phi_e[1]["b"].reshape(1, -1),
    )
    blk_specs = [
        pl.BlockSpec((blk, 8), lambda i: (i, 0)),
        pl.BlockSpec((blk, 8), lambda i: (i, 0)),
        pl.BlockSpec((blk, 4), lambda i: (i, 0)),
        pl.BlockSpec((blk, 1), lambda i: (i, 0)),
    ] + [pl.BlockSpec(a.shape, lambda i: (0, 0)) for a in args[4:]]
    return pl.pallas_call(
        _phi_e_body,
        grid=(grid,),
        in_specs=blk_specs,
        out_specs=[
            pl.BlockSpec((blk, 4), lambda i: (i, 0)),
            pl.BlockSpec((blk, 8), lambda i: (i, 0)),
        ],
        out_shape=[
            jax.ShapeDtypeStruct((e, 4), F32),
            jax.ShapeDtypeStruct((e, 8), F32),
        ],
        interpret=_INTERPRET,
    )(*args)


def _phi_x_body(h_ref, aga_ref, agb_ref,
                xh_ref, xa_ref, bx1_ref, x2_ref, bx2_ref,
                hnext_out):
    h = h_ref[...][:, 0:5]
    agg = aga_ref[...][:, 0:4] + agb_ref[...][:, 0:4]
    blk = h.shape[0]
    t = jax.nn.relu(_mm(h, xh_ref[...]) + _mm(agg, xa_ref[...]) + bx1_ref[...])
    hn = _mm(t, x2_ref[...]) + bx2_ref[...]
    hnext_out[...] = jnp.concatenate([h + hn, jnp.zeros((blk, 3), F32)], axis=1)


def _hc_phi_x(h, agg_a, agg_b, phi_x, blk):
    n = h.shape[0]
    grid = n // blk
    x1 = phi_x[0]["W"]
    args = (
        h, agg_a, agg_b,
        x1[0:5], x1[5:9], phi_x[0]["b"].reshape(1, -1),
        phi_x[1]["W"], phi_x[1]["b"].reshape(1, -1),
    )
    blk_specs = [
        pl.BlockSpec((blk, 8), lambda i: (i, 0)),
        pl.BlockSpec((blk, 8), lambda i: (i, 0)),
        pl.BlockSpec((blk, 8), lambda i: (i, 0)),
    ] + [pl.BlockSpec(a.shape, lambda i: (0, 0)) for a in args[3:]]
    return pl.pallas_call(
        _phi_x_body,
        grid=(grid,),
        in_specs=blk_specs,
        out_specs=pl.BlockSpec((blk, 8), lambda i: (i, 0)),
        out_shape=jax.ShapeDtypeStruct((n, 8), F32),
        interpret=_INTERPRET,
    )(*args)


def _track_edge_body(gs_ref, gd_ref, e0_ref, e1_ref, e2_ref, e3_ref, w_ref,
                     ws_ref, wd_ref, w0_ref, w1_ref, w2_ref, w3_ref, b1_ref,
                     t2_ref, tb2_ref, msg_out):
    hs = gs_ref[...][:, 0:5]
    hd = gd_ref[...][:, 0:5]
    blk = hs.shape[0]
    hid = jax.nn.relu(
        _mm(hs, ws_ref[...]) + _mm(hd, wd_ref[...])
        + _mm(e0_ref[...], w0_ref[...]) + _mm(e1_ref[...], w1_ref[...])
        + _mm(e2_ref[...], w2_ref[...]) + _mm(e3_ref[...], w3_ref[...])
        + b1_ref[...])
    eo = _mm(hid, t2_ref[...]) + tb2_ref[...]
    mask = (w_ref[...] > 0.5).astype(F32)
    msg = eo * mask
    msg_out[...] = jnp.concatenate([msg, jnp.zeros((blk, 7), F32)], axis=1)


def _track_edge(gs, gd, es, w, phi_e, blk):
    e = gs.shape[0]
    grid = e // blk
    w1 = phi_e[0]["W"]
    args = (
        gs, gd, es[0], es[1], es[2], es[3], w,
        w1[0:5], w1[5:10], w1[10:14], w1[14:18], w1[18:22], w1[22:26],
        phi_e[0]["b"].reshape(1, -1),
        phi_e[1]["W"], phi_e[1]["b"].reshape(1, -1),
    )
    blk_specs = [
        pl.BlockSpec((blk, 8), lambda i: (i, 0)),
        pl.BlockSpec((blk, 8), lambda i: (i, 0)),
        pl.BlockSpec((blk, 4), lambda i: (i, 0)),
        pl.BlockSpec((blk, 4), lambda i: (i, 0)),
        pl.BlockSpec((blk, 4), lambda i: (i, 0)),
        pl.BlockSpec((blk, 4), lambda i: (i, 0)),
        pl.BlockSpec((blk, 1), lambda i: (i, 0)),
    ] + [pl.BlockSpec(a.shape, lambda i: (0, 0)) for a in args[7:]]
    return pl.pallas_call(
        _track_edge_body,
        grid=(grid,),
        in_specs=blk_specs,
        out_specs=pl.BlockSpec((blk, 8), lambda i: (i, 0)),
        out_shape=jax.ShapeDtypeStruct((e, 8), F32),
        interpret=_INTERPRET,
    )(*args)


def _heads_body(h_ref, aga_ref, agb_ref,
                b1_ref, bb1_ref, b2_ref, bb2_ref, b3_ref, bb3_ref,
                c1_ref, cb1_ref, c2_ref, cb2_ref, c3_ref, cb3_ref,
                ph_ref, pa_ref, pb1_ref, p2_ref, pb2_ref,
                beta_out, big_h_out, p_out):
    h = h_ref[...][:, 0:5]
    t = jax.nn.relu(_mm(h, b1_ref[...]) + bb1_ref[...])
    t = jax.nn.relu(_mm(t, b2_ref[...]) + bb2_ref[...])
    beta_out[...] = jax.nn.sigmoid(_mm(t, b3_ref[...]) + bb3_ref[...]) + 1e-8
    t = jax.nn.relu(_mm(h, c1_ref[...]) + cb1_ref[...])
    t = jax.nn.relu(_mm(t, c2_ref[...]) + cb2_ref[...])
    big_h_out[...] = _mm(t, c3_ref[...]) + cb3_ref[...]
    agg = aga_ref[...][:, 0:1] + agb_ref[...][:, 0:1]
    t = jax.nn.relu(_mm(h, ph_ref[...]) + _mm(agg, pa_ref[...]) + pb1_ref[...])
    p_out[...] = _mm(t, p2_ref[...]) + pb2_ref[...]


def _heads(h, agg_a, agg_b, p_beta, p_cluster, phi_x, blk):
    n = h.shape[0]
    grid = n // blk
    x1 = phi_x[0]["W"]
    args = (
        h, agg_a, agg_b,
        p_beta[0]["W"], p_beta[0]["b"].reshape(1, -1),
        p_beta[1]["W"], p_beta[1]["b"].reshape(1, -1),
        p_beta[2]["W"], p_beta[2]["b"].reshape(1, -1),
        p_cluster[0]["W"], p_cluster[0]["b"].reshape(1, -1),
        p_cluster[1]["W"], p_cluster[1]["b"].reshape(1, -1),
        p_cluster[2]["W"], p_cluster[2]["b"].reshape(1, -1),
        x1[0:5], x1[5:6], phi_x[0]["b"].reshape(1, -1),
        phi_x[1]["W"], phi_x[1]["b"].reshape(1, -1),
    )
    blk_specs = [
        pl.BlockSpec((blk, 8), lambda i: (i, 0)),
        pl.BlockSpec((blk, 8), lambda i: (i, 0)),
        pl.BlockSpec((blk, 8), lambda i: (i, 0)),
    ] + [pl.BlockSpec(a.shape, lambda i: (0, 0)) for a in args[3:]]
    return pl.pallas_call(
        _heads_body,
        grid=(grid,),
        in_specs=blk_specs,
        out_specs=[
            pl.BlockSpec((blk, 1), lambda i: (i, 0)),
            pl.BlockSpec((blk, 2), lambda i: (i, 0)),
            pl.BlockSpec((blk, 1), lambda i: (i, 0)),
        ],
        out_shape=[
            jax.ShapeDtypeStruct((n, 1), F32),
            jax.ShapeDtypeStruct((n, 2), F32),
            jax.ShapeDtypeStruct((n, 1), F32),
        ],
        interpret=_INTERPRET,
    )(*args)


def _gather_pairs(table, src, dst):
    return jnp.take(table, src, axis=0), jnp.take(table, dst, axis=0)


def _scatter_add(msg8, dst, n):
    agg = jax.ops.segment_sum(msg8, dst, num_segments=n)
    zero = jnp.zeros_like(agg)
    return agg, zero


_BLK_E = 2560
_BLK_N = 2000


def kernel(x, edge_attr, params, edge_index):
    n = x.shape[0]
    src, dst = edge_index[0], edge_index[1]

    h_ec, h = _node_encoders(x, params["ec_node_enc"], params["hc_node_enc"], _BLK_N)
    e_ec, e = _edge_encoders(edge_attr, params["ec_edge_enc"], params["hc_edge_enc"], _BLK_E)

    gs, gd = _gather_pairs(h_ec, src, dst)
    w = _ec_edge(gs, gd, e_ec, params["ec_in"]["phi_e"], params["ec_w"], _BLK_E)

    e_list = [e]
    for layer in params["hc_in"]:
        gs, gd = _gather_pairs(h, src, dst)
        e, msg8 = _hc_phi_e(gs, gd, e, w, layer["phi_e"], _BLK_E)
        agg_a, agg_b = _scatter_add(msg8, dst, n)
        h = _hc_phi_x(h, agg_a, agg_b, layer["phi_x"], _BLK_N)
        e_list.append(e)

    gs, gd = _gather_pairs(h, src, dst)
    msg1 = _track_edge(gs, gd, e_list, w, params["p_track"]["phi_e"], _BLK_E)
    agg1_a, agg1_b = _scatter_add(msg1, dst, n)
    beta, big_h, p_out = _heads(h, agg1_a, agg1_b, params["p_beta"],
                                params["p_cluster"], params["p_track"]["phi_x"],
                                _BLK_N)
    return w, big_h, beta, p_out


# speedup vs baseline: 2.9132x; 2.9132x over previous
import functools

import jax
import jax.numpy as jnp
from jax import lax
from jax.experimental import pallas as pl
from jax.experimental.pallas import tpu as pltpu
from jax.experimental.pallas import tpu_sc as plsc

_INTERPRET = False

F32 = jnp.float32


def _mm(a, b):
    return jnp.dot(a, b, preferred_element_type=F32)


def _node_enc_body(x_ref, wa1_ref, wa2_ref, wb1_ref, wb2_ref, ha_ref, hb_ref):
    x = x_ref[...]
    blk = x.shape[0]
    pad = jnp.zeros((blk, 3), F32)
    ha = jax.nn.relu(_mm(jax.nn.relu(_mm(x, wa1_ref[...])), wa2_ref[...]))
    hb = jax.nn.relu(_mm(jax.nn.relu(_mm(x, wb1_ref[...])), wb2_ref[...]))
    ha_ref[...] = jnp.concatenate([ha, pad], axis=1)
    hb_ref[...] = jnp.concatenate([hb, pad], axis=1)


def _node_encoders(x, p_ec, p_hc, blk):
    n = x.shape[0]
    grid = n // blk
    full = lambda s: pl.BlockSpec(s, lambda i: (0, 0))
    return pl.pallas_call(
        _node_enc_body,
        grid=(grid,),
        in_specs=[
            pl.BlockSpec((blk, x.shape[1]), lambda i: (i, 0)),
            full(p_ec[0]["W"].shape), full(p_ec[1]["W"].shape),
            full(p_hc[0]["W"].shape), full(p_hc[1]["W"].shape),
        ],
        out_specs=[
            pl.BlockSpec((blk, 8), lambda i: (i, 0)),
            pl.BlockSpec((blk, 8), lambda i: (i, 0)),
        ],
        out_shape=[
            jax.ShapeDtypeStruct((n, 8), F32),
            jax.ShapeDtypeStruct((n, 8), F32),
        ],
        interpret=_INTERPRET,
    )(x, p_ec[0]["W"], p_ec[1]["W"], p_hc[0]["W"], p_hc[1]["W"])


def _edge_enc_body(ea_ref, wa1_ref, wa2_ref, wb1_ref, wb2_ref, ea_out, eb_out):
    ea = ea_ref[...]
    ea_out[...] = jax.nn.relu(_mm(jax.nn.relu(_mm(ea, wa1_ref[...])), wa2_ref[...]))
    eb_out[...] = jax.nn.relu(_mm(jax.nn.relu(_mm(ea, wb1_ref[...])), wb2_ref[...]))


def _edge_encoders(edge_attr, p_ec, p_hc, blk):
    e = edge_attr.shape[0]
    grid = e // blk
    full = lambda s: pl.BlockSpec(s, lambda i: (0, 0))
    return pl.pallas_call(
        _edge_enc_body,
        grid=(grid,),
        in_specs=[
            pl.BlockSpec((blk, edge_attr.shape[1]), lambda i: (i, 0)),
            full(p_ec[0]["W"].shape), full(p_ec[1]["W"].shape),
            full(p_hc[0]["W"].shape), full(p_hc[1]["W"].shape),
        ],
        out_specs=[
            pl.BlockSpec((blk, 4), lambda i: (i, 0)),
            pl.BlockSpec((blk, 4), lambda i: (i, 0)),
        ],
        out_shape=[
            jax.ShapeDtypeStruct((e, 4), F32),
            jax.ShapeDtypeStruct((e, 4), F32),
        ],
        interpret=_INTERPRET,
    )(edge_attr, p_ec[0]["W"], p_ec[1]["W"], p_hc[0]["W"], p_hc[1]["W"])


def _ec_edge_body(gs_ref, gd_ref, ee_ref,
                  ws_ref, wd_ref, we_ref, b1_ref, w2_ref, b2_ref,
                  v1_ref, c1_ref, v2_ref, c2_ref, v3_ref, c3_ref,
                  w_out):
    hs = gs_ref[...][:, 0:5]
    hd = gd_ref[...][:, 0:5]
    ee = ee_ref[...]
    hid = jax.nn.relu(_mm(hs, ws_ref[...]) + _mm(hd, wd_ref[...])
                      + _mm(ee, we_ref[...]) + b1_ref[...])
    e2 = _mm(hid, w2_ref[...]) + b2_ref[...]
    t = jax.nn.relu(_mm(e2, v1_ref[...]) + c1_ref[...])
    t = jax.nn.relu(_mm(t, v2_ref[...]) + c2_ref[...])
    w_out[...] = jax.nn.sigmoid(_mm(t, v3_ref[...]) + c3_ref[...])


def _ec_edge(gs, gd, ee, phi_e, ec_w, blk):
    e = ee.shape[0]
    grid = e // blk
    w1 = phi_e[0]["W"]
    args = (
        gs, gd, ee,
        w1[0:5], w1[5:10], w1[10:14], phi_e[0]["b"].reshape(1, -1),
        phi_e[1]["W"], phi_e[1]["b"].reshape(1, -1),
        ec_w[0]["W"], ec_w[0]["b"].reshape(1, -1),
        ec_w[1]["W"], ec_w[1]["b"].reshape(1, -1),
        ec_w[2]["W"], ec_w[2]["b"].reshape(1, -1),
    )
    blk_specs = [
        pl.BlockSpec((blk, 8), lambda i: (i, 0)),
        pl.BlockSpec((blk, 8), lambda i: (i, 0)),
        pl.BlockSpec((blk, 4), lambda i: (i, 0)),
    ] + [pl.BlockSpec(a.shape, lambda i: (0, 0)) for a in args[3:]]
    return pl.pallas_call(
        _ec_edge_body,
        grid=(grid,),
        in_specs=blk_specs,
        out_specs=pl.BlockSpec((blk, 1), lambda i: (i, 0)),
        out_shape=jax.ShapeDtypeStruct((e, 1), F32),
        interpret=_INTERPRET,
    )(*args)


def _phi_e_body(gs_ref, gd_ref, el_ref, w_ref,
                ws_ref, wd_ref, we_ref, b1_ref, w2_ref, b2_ref,
                enext_out, msg_out):
    hs = gs_ref[...][:, 0:5]
    hd = gd_ref[...][:, 0:5]
    el = el_ref[...]
    blk = hs.shape[0]
    hid = jax.nn.relu(_mm(hs, ws_ref[...]) + _mm(hd, wd_ref[...])
                      + _mm(el, we_ref[...]) + b1_ref[...])
    e_new = _mm(hid, w2_ref[...]) + b2_ref[...]
    mask = (w_ref[...] > 0.5).astype(F32)
    msg = e_new * mask
    enext_out[...] = el + e_new
    msg_out[...] = jnp.concatenate([msg, jnp.zeros((blk, 4), F32)], axis=1)


def _hc_phi_e(gs, gd, el, w, phi_e, blk):
    e = el.shape[0]
    grid = e // blk
    w1 = phi_e[0]["W"]
    args = (
        gs, gd, el, w,
        w1[0:5], w1[5:10], w1[10:14], phi_e[0]["b"].reshape(1, -1),
        phi_e[1]["W"], phi_e[1]["b"].reshape(1, -1),
    )
    blk_specs = [
        pl.BlockSpec((blk, 8), lambda i: (i, 0)),
        pl.BlockSpec((blk, 8), lambda i: (i, 0)),
        pl.BlockSpec((blk, 4), lambda i: (i, 0)),
        pl.BlockSpec((blk, 1), lambda i: (i, 0)),
    ] + [pl.BlockSpec(a.shape, lambda i: (0, 0)) for a in args[4:]]
    return pl.pallas_call(
        _phi_e_body,
        grid=(grid,),
        in_specs=blk_specs,
        out_specs=[
            pl.BlockSpec((blk, 4), lambda i: (i, 0)),
            pl.BlockSpec((blk, 8), lambda i: (i, 0)),
        ],
        out_shape=[
            jax.ShapeDtypeStruct((e, 4), F32),
            jax.ShapeDtypeStruct((_E_PAD, 8), F32),
        ],
        interpret=_INTERPRET,
    )(*args)


def _phi_x_body(h_ref, aga_ref, agb_ref,
                xh_ref, xa_ref, bx1_ref, x2_ref, bx2_ref,
                hnext_out):
    h = h_ref[...][:, 0:5]
    agg = aga_ref[...][:, 0:4] + agb_ref[...][:, 0:4]
    blk = h.shape[0]
    t = jax.nn.relu(_mm(h, xh_ref[...]) + _mm(agg, xa_ref[...]) + bx1_ref[...])
    hn = _mm(t, x2_ref[...]) + bx2_ref[...]
    hnext_out[...] = jnp.concatenate([h + hn, jnp.zeros((blk, 3), F32)], axis=1)


def _hc_phi_x(h, agg_a, agg_b, phi_x, blk):
    n = h.shape[0]
    grid = n // blk
    x1 = phi_x[0]["W"]
    args = (
        h, agg_a, agg_b,
        x1[0:5], x1[5:9], phi_x[0]["b"].reshape(1, -1),
        phi_x[1]["W"], phi_x[1]["b"].reshape(1, -1),
    )
    blk_specs = [
        pl.BlockSpec((blk, 8), lambda i: (i, 0)),
        pl.BlockSpec((blk, 8), lambda i: (i, 0)),
        pl.BlockSpec((blk, 8), lambda i: (i, 0)),
    ] + [pl.BlockSpec(a.shape, lambda i: (0, 0)) for a in args[3:]]
    return pl.pallas_call(
        _phi_x_body,
        grid=(grid,),
        in_specs=blk_specs,
        out_specs=pl.BlockSpec((blk, 8), lambda i: (i, 0)),
        out_shape=jax.ShapeDtypeStruct((n, 8), F32),
        interpret=_INTERPRET,
    )(*args)


def _track_edge_body(gs_ref, gd_ref, e0_ref, e1_ref, e2_ref, e3_ref, w_ref,
                     ws_ref, wd_ref, w0_ref, w1_ref, w2_ref, w3_ref, b1_ref,
                     t2_ref, tb2_ref, msg_out):
    hs = gs_ref[...][:, 0:5]
    hd = gd_ref[...][:, 0:5]
    blk = hs.shape[0]
    hid = jax.nn.relu(
        _mm(hs, ws_ref[...]) + _mm(hd, wd_ref[...])
        + _mm(e0_ref[...], w0_ref[...]) + _mm(e1_ref[...], w1_ref[...])
        + _mm(e2_ref[...], w2_ref[...]) + _mm(e3_ref[...], w3_ref[...])
        + b1_ref[...])
    eo = _mm(hid, t2_ref[...]) + tb2_ref[...]
    mask = (w_ref[...] > 0.5).astype(F32)
    msg = eo * mask
    msg_out[...] = jnp.concatenate([msg, jnp.zeros((blk, 7), F32)], axis=1)


def _track_edge(gs, gd, es, w, phi_e, blk):
    e = es[0].shape[0]
    grid = e // blk
    w1 = phi_e[0]["W"]
    args = (
        gs, gd, es[0], es[1], es[2], es[3], w,
        w1[0:5], w1[5:10], w1[10:14], w1[14:18], w1[18:22], w1[22:26],
        phi_e[0]["b"].reshape(1, -1),
        phi_e[1]["W"], phi_e[1]["b"].reshape(1, -1),
    )
    blk_specs = [
        pl.BlockSpec((blk, 8), lambda i: (i, 0)),
        pl.BlockSpec((blk, 8), lambda i: (i, 0)),
        pl.BlockSpec((blk, 4), lambda i: (i, 0)),
        pl.BlockSpec((blk, 4), lambda i: (i, 0)),
        pl.BlockSpec((blk, 4), lambda i: (i, 0)),
        pl.BlockSpec((blk, 4), lambda i: (i, 0)),
        pl.BlockSpec((blk, 1), lambda i: (i, 0)),
    ] + [pl.BlockSpec(a.shape, lambda i: (0, 0)) for a in args[7:]]
    return pl.pallas_call(
        _track_edge_body,
        grid=(grid,),
        in_specs=blk_specs,
        out_specs=pl.BlockSpec((blk, 8), lambda i: (i, 0)),
        out_shape=jax.ShapeDtypeStruct((_E_PAD, 8), F32),
        interpret=_INTERPRET,
    )(*args)


def _heads_body(h_ref, aga_ref, agb_ref,
                b1_ref, bb1_ref, b2_ref, bb2_ref, b3_ref, bb3_ref,
                c1_ref, cb1_ref, c2_ref, cb2_ref, c3_ref, cb3_ref,
                ph_ref, pa_ref, pb1_ref, p2_ref, pb2_ref,
                beta_out, big_h_out, p_out):
    h = h_ref[...][:, 0:5]
    t = jax.nn.relu(_mm(h, b1_ref[...]) + bb1_ref[...])
    t = jax.nn.relu(_mm(t, b2_ref[...]) + bb2_ref[...])
    beta_out[...] = jax.nn.sigmoid(_mm(t, b3_ref[...]) + bb3_ref[...]) + 1e-8
    t = jax.nn.relu(_mm(h, c1_ref[...]) + cb1_ref[...])
    t = jax.nn.relu(_mm(t, c2_ref[...]) + cb2_ref[...])
    big_h_out[...] = _mm(t, c3_ref[...]) + cb3_ref[...]
    agg = aga_ref[...][:, 0:1] + agb_ref[...][:, 0:1]
    t = jax.nn.relu(_mm(h, ph_ref[...]) + _mm(agg, pa_ref[...]) + pb1_ref[...])
    p_out[...] = _mm(t, p2_ref[...]) + pb2_ref[...]


def _heads(h, agg_a, agg_b, p_beta, p_cluster, phi_x, blk):
    n = h.shape[0]
    grid = n // blk
    x1 = phi_x[0]["W"]
    args = (
        h, agg_a, agg_b,
        p_beta[0]["W"], p_beta[0]["b"].reshape(1, -1),
        p_beta[1]["W"], p_beta[1]["b"].reshape(1, -1),
        p_beta[2]["W"], p_beta[2]["b"].reshape(1, -1),
        p_cluster[0]["W"], p_cluster[0]["b"].reshape(1, -1),
        p_cluster[1]["W"], p_cluster[1]["b"].reshape(1, -1),
        p_cluster[2]["W"], p_cluster[2]["b"].reshape(1, -1),
        x1[0:5], x1[5:6], phi_x[0]["b"].reshape(1, -1),
        phi_x[1]["W"], phi_x[1]["b"].reshape(1, -1),
    )
    blk_specs = [
        pl.BlockSpec((blk, 8), lambda i: (i, 0)),
        pl.BlockSpec((blk, 8), lambda i: (i, 0)),
        pl.BlockSpec((blk, 8), lambda i: (i, 0)),
    ] + [pl.BlockSpec(a.shape, lambda i: (0, 0)) for a in args[3:]]
    return pl.pallas_call(
        _heads_body,
        grid=(grid,),
        in_specs=blk_specs,
        out_specs=[
            pl.BlockSpec((blk, 1), lambda i: (i, 0)),
            pl.BlockSpec((blk, 2), lambda i: (i, 0)),
            pl.BlockSpec((blk, 1), lambda i: (i, 0)),
        ],
        out_shape=[
            jax.ShapeDtypeStruct((n, 1), F32),
            jax.ShapeDtypeStruct((n, 2), F32),
            jax.ShapeDtypeStruct((n, 1), F32),
        ],
        interpret=_INTERPRET,
    )(*args)


_NC, _NS = 2, 16
_NW = _NC * _NS
_CH = 128
_E = 320000
_E_PAD = 327680
_N_MAIN = _E_PAD // (_NW * _CH)
_N_PAD = 10240


def _sc_mesh():
    return plsc.VectorSubcoreMesh(core_axis_name="c", subcore_axis_name="s",
                                  num_cores=_NC, num_subcores=_NS)


def _fire_groups(n_chunks, group, fire_one):
    n_groups = n_chunks // group

    def body(g, _):
        ds = [fire_one(g * group + jj) for jj in range(group)]
        for d in ds:
            d.wait()
        return 0

    lax.fori_loop(0, n_groups, body, 0)


def _gather_body(table, src2d, dst2d, gs, gd, idx2d, rows2d, sem):
    c = lax.axis_index("c")
    s = lax.axis_index("s")
    t = s * _NC + c
    base_chunk = t * _N_MAIN
    base_edge = base_chunk * _CH

    for idx_hbm, out_hbm in ((src2d, gs), (dst2d, gd)):
        pltpu.sync_copy(idx_hbm.at[pl.ds(base_chunk, _N_MAIN)], idx2d)

        def fire(j):
            return pltpu.async_copy(
                table.at[idx2d.at[j]],
                rows2d.at[pl.ds(j * _CH, _CH)], sem)

        _fire_groups(_N_MAIN, 8, fire)
        pltpu.sync_copy(rows2d, out_hbm.at[pl.ds(base_edge, _N_MAIN * _CH)])


def _gather_pairs(table, src2d, dst2d):
    gather = pl.kernel(
        _gather_body,
        compiler_params=pltpu.CompilerParams(use_tc_tiling_on_sc=False),
        out_type=[
            jax.ShapeDtypeStruct((_E_PAD, 8), F32),
            jax.ShapeDtypeStruct((_E_PAD, 8), F32),
        ],
        mesh=_sc_mesh(),
        scratch_types=[
            pltpu.VMEM((_N_MAIN, _CH), jnp.int32),
            pltpu.VMEM((_N_MAIN * _CH, 8), F32),
            pltpu.SemaphoreType.DMA,
        ],
    )
    return gather(table, src2d, dst2d)


def _scatter_body(msg, dst2d, zeros, agg_a, agg_b, acc, idx2d, msgbuf, sem):
    c = lax.axis_index("c")
    s = lax.axis_index("s")
    t = s * _NC + c
    base_chunk = t * _N_MAIN
    base_edge = base_chunk * _CH
    rows_per_sub = _N_PAD // _NS

    pltpu.sync_copy(zeros.at[pl.ds(s * rows_per_sub, rows_per_sub)],
                    acc.at[pl.ds(s * rows_per_sub, rows_per_sub)])
    plsc.subcore_barrier()

    pltpu.sync_copy(dst2d.at[pl.ds(base_chunk, _N_MAIN)], idx2d)
    pltpu.sync_copy(msg.at[pl.ds(base_edge, _N_MAIN * _CH)], msgbuf)

    def fire(j):
        return pltpu.async_copy(
            msgbuf.at[pl.ds(j * _CH, _CH)],
            acc.at[idx2d.at[j]], sem, add=True)

    _fire_groups(_N_MAIN, 8, fire)
    plsc.subcore_barrier()

    @pl.when(c == 0)
    def _():
        pltpu.sync_copy(acc.at[pl.ds(s * rows_per_sub, rows_per_sub)],
                        agg_a.at[pl.ds(s * rows_per_sub, rows_per_sub)])

    @pl.when(c == 1)
    def _():
        pltpu.sync_copy(acc.at[pl.ds(s * rows_per_sub, rows_per_sub)],
                        agg_b.at[pl.ds(s * rows_per_sub, rows_per_sub)])


def _scatter_add_jnp(msg8, dst2d, zeros):
    dst = dst2d.reshape(-1)
    agg = jax.ops.segment_sum(msg8, dst, num_segments=zeros.shape[0])
    return agg, zeros


def _scatter_add(msg8, dst2d, zeros):
    scatter = pl.kernel(
        _scatter_body,
        compiler_params=pltpu.CompilerParams(use_tc_tiling_on_sc=False),
        out_type=[
            jax.ShapeDtypeStruct((_N_PAD, 8), F32),
            jax.ShapeDtypeStruct((_N_PAD, 8), F32),
        ],
        mesh=_sc_mesh(),
        scratch_types=[
            pltpu.VMEM_SHARED((_N_PAD, 8), F32),
            pltpu.VMEM((_N_MAIN, _CH), jnp.int32),
            pltpu.VMEM((_N_MAIN * _CH, 8), F32),
            pltpu.SemaphoreType.DMA,
        ],
    )
    return scatter(msg8, dst2d, zeros)


_BLK_E = 2560
_BLK_N = 2000


def kernel(x, edge_attr, params, edge_index):
    n = x.shape[0]
    e_cnt = edge_index.shape[1]
    pad = _E_PAD - e_cnt
    src_pad = jnp.concatenate([edge_index[0], jnp.zeros((pad,), jnp.int32)])
    dst_pad = jnp.concatenate([edge_index[1], jnp.full((pad,), n, jnp.int32)])
    src2d = src_pad.reshape(_E_PAD // _CH, _CH)
    dst2d = dst_pad.reshape(_E_PAD // _CH, _CH)
    zeros = jnp.zeros((_N_PAD, 8), F32)

    h_ec, h = _node_encoders(x, params["ec_node_enc"], params["hc_node_enc"], _BLK_N)
    e_ec, e = _edge_encoders(edge_attr, params["ec_edge_enc"], params["hc_edge_enc"], _BLK_E)

    gs, gd = _gather_pairs(h_ec, src2d, dst2d)
    w = _ec_edge(gs, gd, e_ec, params["ec_in"]["phi_e"], params["ec_w"], _BLK_E)

    e_list = [e]
    for layer in params["hc_in"]:
        gs, gd = _gather_pairs(h, src2d, dst2d)
        e, msg8 = _hc_phi_e(gs, gd, e, w, layer["phi_e"], _BLK_E)
        agg_a, agg_b = _scatter_add(msg8, dst2d, zeros)
        h = _hc_phi_x(h, agg_a, agg_b, layer["phi_x"], _BLK_N)
        e_list.append(e)

    gs, gd = _gather_pairs(h, src2d, dst2d)
    msg1 = _track_edge(gs, gd, e_list, w, params["p_track"]["phi_e"], _BLK_E)
    agg1_a, agg1_b = _scatter_add(msg1, dst2d, zeros)
    beta, big_h, p_out = _heads(h, agg1_a, agg1_b, params["p_beta"],
                                params["p_cluster"], params["p_track"]["phi_x"],
                                _BLK_N)
    return w, big_h, beta, p_out


# speedup vs baseline: 6.0972x; 2.0930x over previous
import functools

import jax
import jax.numpy as jnp
from jax import lax
from jax.experimental import pallas as pl
from jax.experimental.pallas import tpu as pltpu
from jax.experimental.pallas import tpu_sc as plsc

_INTERPRET = False

F32 = jnp.float32

_NC, _NS = 2, 16
_NW = _NC * _NS
_CH = 128
_E = 320000
_E_PAD = 327680
_N_MAIN = _E_PAD // (_NW * _CH)
_N_PAD = 10240
_R = _E_PAD * 8 // 128
_RB = 160
_BLK_N = 2000


def _mm(a, b):
    return jnp.dot(a, b, preferred_element_type=F32)


_EYE16 = None


def _eye16():
    return jnp.eye(16, dtype=F32)


def _pack_in(w):
    wp = jnp.zeros((8, 40), F32).at[: w.shape[0]].set(w)
    return (_eye16()[:, None, :, None] * wp[None, :, None, :]).reshape(128, 640)


def _pack_out(w):
    wp = jnp.zeros((40, 8), F32).at[:, : w.shape[1]].set(w)
    return (_eye16()[:, None, :, None] * wp[None, :, None, :]).reshape(640, 128)


def _pack_out_rep(w):
    return _pack_out(jnp.tile(w, (1, 8)))


def _pack_mid(w):
    return (_eye16()[:, None, :, None] * w[None, :, None, :]).reshape(640, 640)


def _tile_b(b):
    return jnp.tile(b, 16).reshape(1, 640)


def _tile_b8(b):
    bp = jnp.zeros((8,), F32).at[: b.shape[0]].set(b)
    return jnp.tile(bp, 16).reshape(1, 128)


def _tile_b8_rep(b):
    return jnp.tile(jnp.full((8,), b[0], F32), 16).reshape(1, 128)


def _full(a):
    return pl.BlockSpec(a.shape, lambda i: (0, 0))


def _pblk():
    return pl.BlockSpec((_RB, 128), lambda i: (i, 0))


def _node_enc_body(x_ref, wa1_ref, wa2_ref, wb1_ref, wb2_ref, ha_ref, hb_ref):
    x = x_ref[...]
    blk = x.shape[0]
    pad = jnp.zeros((blk, 3), F32)
    ha = jax.nn.relu(_mm(jax.nn.relu(_mm(x, wa1_ref[...])), wa2_ref[...]))
    hb = jax.nn.relu(_mm(jax.nn.relu(_mm(x, wb1_ref[...])), wb2_ref[...]))
    ha_ref[...] = jnp.concatenate([ha, pad], axis=1)
    hb_ref[...] = jnp.concatenate([hb, pad], axis=1)


def _node_encoders(x, p_ec, p_hc, blk):
    n = x.shape[0]
    return pl.pallas_call(
        _node_enc_body,
        grid=(n // blk,),
        in_specs=[
            pl.BlockSpec((blk, x.shape[1]), lambda i: (i, 0)),
            _full(p_ec[0]["W"]), _full(p_ec[1]["W"]),
            _full(p_hc[0]["W"]), _full(p_hc[1]["W"]),
        ],
        out_specs=[
            pl.BlockSpec((blk, 8), lambda i: (i, 0)),
            pl.BlockSpec((blk, 8), lambda i: (i, 0)),
        ],
        out_shape=[
            jax.ShapeDtypeStruct((n, 8), F32),
            jax.ShapeDtypeStruct((n, 8), F32),
        ],
        interpret=_INTERPRET,
    )(x, p_ec[0]["W"], p_ec[1]["W"], p_hc[0]["W"], p_hc[1]["W"])


def _edge_enc_body(ea_ref, wa1_ref, wa2_ref, wb1_ref, wb2_ref, ea_out, eb_out):
    ea = ea_ref[...]
    ea_out[...] = jax.nn.relu(_mm(jax.nn.relu(_mm(ea, wa1_ref[...])), wa2_ref[...]))
    eb_out[...] = jax.nn.relu(_mm(jax.nn.relu(_mm(ea, wb1_ref[...])), wb2_ref[...]))


def _edge_encoders(eap, p_ec, p_hc):
    args = (
        eap,
        _pack_in(p_ec[0]["W"]), _pack_out(p_ec[1]["W"]),
        _pack_in(p_hc[0]["W"]), _pack_out(p_hc[1]["W"]),
    )
    return pl.pallas_call(
        _edge_enc_body,
        grid=(_R // _RB,),
        in_specs=[_pblk()] + [_full(a) for a in args[1:]],
        out_specs=[_pblk(), _pblk()],
        out_shape=[
            jax.ShapeDtypeStruct((_R, 128), F32),
            jax.ShapeDtypeStruct((_R, 128), F32),
        ],
        interpret=_INTERPRET,
    )(*args)


def _ec_edge_body(gs_ref, gd_ref, ee_ref,
                  ws_ref, wd_ref, we_ref, b1_ref, w2_ref, b2_ref,
                  v1_ref, c1_ref, v2_ref, c2_ref, v3_ref, c3_ref,
                  w_out):
    hid = jax.nn.relu(_mm(gs_ref[...], ws_ref[...]) + _mm(gd_ref[...], wd_ref[...])
                      + _mm(ee_ref[...], we_ref[...]) + b1_ref[...])
    e2 = _mm(hid, w2_ref[...]) + b2_ref[...]
    t = jax.nn.relu(_mm(e2, v1_ref[...]) + c1_ref[...])
    t = jax.nn.relu(_mm(t, v2_ref[...]) + c2_ref[...])
    w_out[...] = jax.nn.sigmoid(_mm(t, v3_ref[...]) + c3_ref[...])


def _ec_edge(gs, gd, ee, phi_e, ec_w):
    w1 = phi_e[0]["W"]
    args = (
        gs, gd, ee,
        _pack_in(w1[0:5]), _pack_in(w1[5:10]), _pack_in(w1[10:14]),
        _tile_b(phi_e[0]["b"]),
        _pack_out(phi_e[1]["W"]), _tile_b8(phi_e[1]["b"]),
        _pack_in(ec_w[0]["W"]), _tile_b(ec_w[0]["b"]),
        _pack_mid(ec_w[1]["W"]), _tile_b(ec_w[1]["b"]),
        _pack_out_rep(ec_w[2]["W"]), _tile_b8_rep(ec_w[2]["b"]),
    )
    return pl.pallas_call(
        _ec_edge_body,
        grid=(_R // _RB,),
        in_specs=[_pblk()] * 3 + [_full(a) for a in args[3:]],
        out_specs=_pblk(),
        out_shape=jax.ShapeDtypeStruct((_R, 128), F32),
        interpret=_INTERPRET,
    )(*args)


def _phi_e_body(gs_ref, gd_ref, el_ref, wp_ref,
                ws_ref, wd_ref, we_ref, b1_ref, w2_ref, b2_ref,
                enext_out, msg_out):
    el = el_ref[...]
    hid = jax.nn.relu(_mm(gs_ref[...], ws_ref[...]) + _mm(gd_ref[...], wd_ref[...])
                      + _mm(el, we_ref[...]) + b1_ref[...])
    e_new = _mm(hid, w2_ref[...]) + b2_ref[...]
    mask = (wp_ref[...] > 0.5).astype(F32)
    msg_out[...] = e_new * mask
    enext_out[...] = el + e_new


def _hc_phi_e(gs, gd, el, wp, phi_e):
    w1 = phi_e[0]["W"]
    args = (
        gs, gd, el, wp,
        _pack_in(w1[0:5]), _pack_in(w1[5:10]), _pack_in(w1[10:14]),
        _tile_b(phi_e[0]["b"]),
        _pack_out(phi_e[1]["W"]), _tile_b8(phi_e[1]["b"]),
    )
    return pl.pallas_call(
        _phi_e_body,
        grid=(_R // _RB,),
        in_specs=[_pblk()] * 4 + [_full(a) for a in args[4:]],
        out_specs=[_pblk(), _pblk()],
        out_shape=[
            jax.ShapeDtypeStruct((_R, 128), F32),
            jax.ShapeDtypeStruct((_R, 128), F32),
        ],
        interpret=_INTERPRET,
    )(*args)


def _track_edge_body(gs_ref, gd_ref, e0_ref, e1_ref, e2_ref, e3_ref, wp_ref,
                     ws_ref, wd_ref, w0_ref, w1_ref, w2_ref, w3_ref, b1_ref,
                     t2_ref, tb2_ref, msg_out):
    hid = jax.nn.relu(
        _mm(gs_ref[...], ws_ref[...]) + _mm(gd_ref[...], wd_ref[...])
        + _mm(e0_ref[...], w0_ref[...]) + _mm(e1_ref[...], w1_ref[...])
        + _mm(e2_ref[...], w2_ref[...]) + _mm(e3_ref[...], w3_ref[...])
        + b1_ref[...])
    eo = _mm(hid, t2_ref[...]) + tb2_ref[...]
    mask = (wp_ref[...] > 0.5).astype(F32)
    msg_out[...] = eo * mask


def _track_edge(gs, gd, es, wp, phi_e):
    w1 = phi_e[0]["W"]
    args = (
        gs, gd, es[0], es[1], es[2], es[3], wp,
        _pack_in(w1[0:5]), _pack_in(w1[5:10]),
        _pack_in(w1[10:14]), _pack_in(w1[14:18]),
        _pack_in(w1[18:22]), _pack_in(w1[22:26]),
        _tile_b(phi_e[0]["b"]),
        _pack_out_rep(phi_e[1]["W"]), _tile_b8_rep(phi_e[1]["b"]),
    )
    return pl.pallas_call(
        _track_edge_body,
        grid=(_R // _RB,),
        in_specs=[_pblk()] * 7 + [_full(a) for a in args[7:]],
        out_specs=_pblk(),
        out_shape=jax.ShapeDtypeStruct((_R, 128), F32),
        interpret=_INTERPRET,
    )(*args)


def _phi_x_body(h_ref, aga_ref, agb_ref,
                xh_ref, xa_ref, bx1_ref, x2_ref, bx2_ref,
                hnext_out):
    h = h_ref[...][:, 0:5]
    agg = aga_ref[...][:, 0:4] + agb_ref[...][:, 0:4]
    blk = h.shape[0]
    t = jax.nn.relu(_mm(h, xh_ref[...]) + _mm(agg, xa_ref[...]) + bx1_ref[...])
    hn = _mm(t, x2_ref[...]) + bx2_ref[...]
    hnext_out[...] = jnp.concatenate([h + hn, jnp.zeros((blk, 3), F32)], axis=1)


def _hc_phi_x(h, agg_a, agg_b, phi_x, blk):
    n = h.shape[0]
    x1 = phi_x[0]["W"]
    args = (
        h, agg_a, agg_b,
        x1[0:5], x1[5:9], phi_x[0]["b"].reshape(1, -1),
        phi_x[1]["W"], phi_x[1]["b"].reshape(1, -1),
    )
    blk_specs = [
        pl.BlockSpec((blk, 8), lambda i: (i, 0)),
        pl.BlockSpec((blk, 8), lambda i: (i, 0)),
        pl.BlockSpec((blk, 8), lambda i: (i, 0)),
    ] + [_full(a) for a in args[3:]]
    return pl.pallas_call(
        _phi_x_body,
        grid=(n // blk,),
        in_specs=blk_specs,
        out_specs=pl.BlockSpec((blk, 8), lambda i: (i, 0)),
        out_shape=jax.ShapeDtypeStruct((n, 8), F32),
        interpret=_INTERPRET,
    )(*args)


def _heads_body(h_ref, aga_ref, agb_ref,
                b1_ref, bb1_ref, b2_ref, bb2_ref, b3_ref, bb3_ref,
                c1_ref, cb1_ref, c2_ref, cb2_ref, c3_ref, cb3_ref,
                ph_ref, pa_ref, pb1_ref, p2_ref, pb2_ref,
                beta_out, big_h_out, p_out):
    h = h_ref[...][:, 0:5]
    t = jax.nn.relu(_mm(h, b1_ref[...]) + bb1_ref[...])
    t = jax.nn.relu(_mm(t, b2_ref[...]) + bb2_ref[...])
    beta_out[...] = jax.nn.sigmoid(_mm(t, b3_ref[...]) + bb3_ref[...]) + 1e-8
    t = jax.nn.relu(_mm(h, c1_ref[...]) + cb1_ref[...])
    t = jax.nn.relu(_mm(t, c2_ref[...]) + cb2_ref[...])
    big_h_out[...] = _mm(t, c3_ref[...]) + cb3_ref[...]
    agg = aga_ref[...][:, 0:1] + agb_ref[...][:, 0:1]
    t = jax.nn.relu(_mm(h, ph_ref[...]) + _mm(agg, pa_ref[...]) + pb1_ref[...])
    p_out[...] = _mm(t, p2_ref[...]) + pb2_ref[...]


def _heads(h, agg_a, agg_b, p_beta, p_cluster, phi_x, blk):
    n = h.shape[0]
    x1 = phi_x[0]["W"]
    args = (
        h, agg_a, agg_b,
        p_beta[0]["W"], p_beta[0]["b"].reshape(1, -1),
        p_beta[1]["W"], p_beta[1]["b"].reshape(1, -1),
        p_beta[2]["W"], p_beta[2]["b"].reshape(1, -1),
        p_cluster[0]["W"], p_cluster[0]["b"].reshape(1, -1),
        p_cluster[1]["W"], p_cluster[1]["b"].reshape(1, -1),
        p_cluster[2]["W"], p_cluster[2]["b"].reshape(1, -1),
        x1[0:5], x1[5:6], phi_x[0]["b"].reshape(1, -1),
        phi_x[1]["W"], phi_x[1]["b"].reshape(1, -1),
    )
    blk_specs = [
        pl.BlockSpec((blk, 8), lambda i: (i, 0)),
        pl.BlockSpec((blk, 8), lambda i: (i, 0)),
        pl.BlockSpec((blk, 8), lambda i: (i, 0)),
    ] + [_full(a) for a in args[3:]]
    return pl.pallas_call(
        _heads_body,
        grid=(n // blk,),
        in_specs=blk_specs,
        out_specs=[
            pl.BlockSpec((blk, 1), lambda i: (i, 0)),
            pl.BlockSpec((blk, 2), lambda i: (i, 0)),
            pl.BlockSpec((blk, 1), lambda i: (i, 0)),
        ],
        out_shape=[
            jax.ShapeDtypeStruct((n, 1), F32),
            jax.ShapeDtypeStruct((n, 2), F32),
            jax.ShapeDtypeStruct((n, 1), F32),
        ],
        interpret=_INTERPRET,
    )(*args)


def _sc_mesh():
    return plsc.VectorSubcoreMesh(core_axis_name="c", subcore_axis_name="s",
                                  num_cores=_NC, num_subcores=_NS)


def _fire_groups(n_chunks, group, fire_one):
    n_groups = n_chunks // group

    def body(g, _):
        ds = [fire_one(g * group + jj) for jj in range(group)]
        for d in ds:
            d.wait()
        return 0

    lax.fori_loop(0, n_groups, body, 0)


def _gather_body(table, src2d, dst2d, gs, gd, idx2d, rows2d, sem):
    c = lax.axis_index("c")
    s = lax.axis_index("s")
    t = s * _NC + c
    base_chunk = t * _N_MAIN
    base_edge = base_chunk * _CH

    for idx_hbm, out_hbm in ((src2d, gs), (dst2d, gd)):
        pltpu.sync_copy(idx_hbm.at[pl.ds(base_chunk, _N_MAIN)], idx2d)

        def fire(j):
            return pltpu.async_copy(
                table.at[idx2d.at[j]],
                rows2d.at[pl.ds(j * _CH, _CH)], sem)

        _fire_groups(_N_MAIN, 8, fire)
        pltpu.sync_copy(rows2d, out_hbm.at[pl.ds(base_edge, _N_MAIN * _CH)])


def _gather_pairs(table, src2d, dst2d):
    gather = pl.kernel(
        _gather_body,
        compiler_params=pltpu.CompilerParams(use_tc_tiling_on_sc=False),
        out_type=[
            jax.ShapeDtypeStruct((_E_PAD, 8), F32),
            jax.ShapeDtypeStruct((_E_PAD, 8), F32),
        ],
        mesh=_sc_mesh(),
        scratch_types=[
            pltpu.VMEM((_N_MAIN, _CH), jnp.int32),
            pltpu.VMEM((_N_MAIN * _CH, 8), F32),
            pltpu.SemaphoreType.DMA,
        ],
    )
    gs, gd = gather(table, src2d, dst2d)
    return gs.reshape(_R, 128), gd.reshape(_R, 128)


def _pack_ea(edge_attr, zeros):
    del zeros
    e_cnt = edge_attr.shape[0]
    eap = jnp.pad(edge_attr, ((0, _E_PAD - e_cnt), (0, 4)))
    return eap.reshape(_R, 128)


def _scatter_body(msg, dst2d, zeros, agg_a, agg_b, acc, idx2d, msgbuf, sem):
    c = lax.axis_index("c")
    s = lax.axis_index("s")
    t = s * _NC + c
    base_chunk = t * _N_MAIN
    base_edge = base_chunk * _CH
    rows_per_sub = _N_PAD // _NS

    pltpu.sync_copy(zeros.at[pl.ds(s * rows_per_sub, rows_per_sub)],
                    acc.at[pl.ds(s * rows_per_sub, rows_per_sub)])
    plsc.subcore_barrier()

    pltpu.sync_copy(dst2d.at[pl.ds(base_chunk, _N_MAIN)], idx2d)
    pltpu.sync_copy(msg.at[pl.ds(base_edge, _N_MAIN * _CH)], msgbuf)

    def fire(j):
        return pltpu.async_copy(
            msgbuf.at[pl.ds(j * _CH, _CH)],
            acc.at[idx2d.at[j]], sem, add=True)

    _fire_groups(_N_MAIN, 8, fire)
    plsc.subcore_barrier()

    @pl.when(c == 0)
    def _():
        pltpu.sync_copy(acc.at[pl.ds(s * rows_per_sub, rows_per_sub)],
                        agg_a.at[pl.ds(s * rows_per_sub, rows_per_sub)])

    @pl.when(c == 1)
    def _():
        pltpu.sync_copy(acc.at[pl.ds(s * rows_per_sub, rows_per_sub)],
                        agg_b.at[pl.ds(s * rows_per_sub, rows_per_sub)])


def _scatter_add(msg_p, dst2d, zeros):
    scatter = pl.kernel(
        _scatter_body,
        compiler_params=pltpu.CompilerParams(use_tc_tiling_on_sc=False),
        out_type=[
            jax.ShapeDtypeStruct((_N_PAD, 8), F32),
            jax.ShapeDtypeStruct((_N_PAD, 8), F32),
        ],
        mesh=_sc_mesh(),
        scratch_types=[
            pltpu.VMEM_SHARED((_N_PAD, 8), F32),
            pltpu.VMEM((_N_MAIN, _CH), jnp.int32),
            pltpu.VMEM((_N_MAIN * _CH, 8), F32),
            pltpu.SemaphoreType.DMA,
        ],
    )
    return scatter(msg_p.reshape(_E_PAD, 8), dst2d, zeros)


def kernel(x, edge_attr, params, edge_index):
    n = x.shape[0]
    e_cnt = edge_index.shape[1]
    pad = _E_PAD - e_cnt
    src_pad = jnp.concatenate([edge_index[0], jnp.zeros((pad,), jnp.int32)])
    dst_pad = jnp.concatenate([edge_index[1], jnp.full((pad,), n, jnp.int32)])
    src2d = src_pad.reshape(_E_PAD // _CH, _CH)
    dst2d = dst_pad.reshape(_E_PAD // _CH, _CH)
    zeros = jnp.zeros((_N_PAD, 8), F32)

    h_ec, h = _node_encoders(x, params["ec_node_enc"], params["hc_node_enc"], _BLK_N)
    eap = _pack_ea(edge_attr, zeros)
    e_ec, e = _edge_encoders(eap, params["ec_edge_enc"], params["hc_edge_enc"])

    gs, gd = _gather_pairs(h_ec, src2d, dst2d)
    wp = _ec_edge(gs, gd, e_ec, params["ec_in"]["phi_e"], params["ec_w"])

    e_list = [e]
    for layer in params["hc_in"]:
        gs, gd = _gather_pairs(h, src2d, dst2d)
        e, msg_p = _hc_phi_e(gs, gd, e, wp, layer["phi_e"])
        agg_a, agg_b = _scatter_add(msg_p, dst2d, zeros)
        h = _hc_phi_x(h, agg_a, agg_b, layer["phi_x"], _BLK_N)
        e_list.append(e)

    gs, gd = _gather_pairs(h, src2d, dst2d)
    msg1 = _track_edge(gs, gd, e_list, wp, params["p_track"]["phi_e"])
    agg1_a, agg1_b = _scatter_add(msg1, dst2d, zeros)
    beta, big_h, p_out = _heads(h, agg1_a, agg1_b, params["p_beta"],
                                params["p_cluster"], params["p_track"]["phi_x"],
                                _BLK_N)
    w = wp.reshape(_E_PAD, 8)[:e_cnt, 0:1]
    return w, big_h, beta, p_out
